# baseline (speedup 1.0000x reference)
import functools

import jax
import jax.numpy as jnp
from jax import lax
from jax.experimental import pallas as pl
from jax.experimental.pallas import tpu as pltpu
from jax.experimental.pallas import tpu_sc as plsc

B = 16384
F = 26
V = 100000
D = 32

NC = 2
NS = 16
NW = NC * NS
BPW = B // NW
CH = 128
NJ = BPW // CH
L = 16


def _body(ftab_hbm, idx_hbm, out_hbm, idx_v, buf_v, acc_v, sem):
    c = lax.axis_index("c")
    s = lax.axis_index("s")
    wid = s * NC + c

    pltpu.sync_copy(idx_hbm.at[wid], idx_v)

    def zstep(i, _):
        acc_v[pl.ds(i * L, L)] = jnp.zeros((L,), jnp.float32)
        return 0
    lax.fori_loop(0, BPW * D // L, zstep, 0)

    def ostep(i, _):
        f = i // (BPW // L)
        r = i - f * (BPW // L)
        j = r // (CH // L)
        cc = r - j * (CH // L)
        sl = idx_v.at[f, j, pl.ds(cc * L, L)]
        sl[...] = sl[...] + f * V
        return 0
    lax.fori_loop(0, F * BPW // L, ostep, 0)

    def gstep(i, _):
        f = i // NJ
        j = i - f * NJ
        pltpu.async_copy(ftab_hbm.at[idx_v.at[f, j]], buf_v, sem).wait()

        def astep(r, _):
            for h in range(D // L):
                off = j * CH * D + r * D + h * L
                plsc.addupdate(acc_v.at[pl.ds(off, L)],
                               buf_v[r, pl.ds(h * L, L)])
            return 0
        lax.fori_loop(0, CH, astep, 0)
        return 0
    lax.fori_loop(0, F * NJ, gstep, 0)

    pltpu.sync_copy(acc_v, out_hbm.at[pl.ds(wid * BPW * D, BPW * D)])


@jax.jit
def _encode(ftab, idx4):
    mesh = plsc.VectorSubcoreMesh(
        core_axis_name="c", subcore_axis_name="s",
        num_cores=NC, num_subcores=NS)
    fn = pl.kernel(
        _body,
        out_type=jax.ShapeDtypeStruct((B * D,), jnp.float32),
        mesh=mesh,
        scratch_types=[
            pltpu.VMEM((F, NJ, CH), jnp.int32),
            pltpu.VMEM((CH, D), jnp.float32),
            pltpu.VMEM((BPW * D,), jnp.float32),
            pltpu.SemaphoreType.DMA,
        ],
        compiler_params=pltpu.CompilerParams(use_tc_tiling_on_sc=False),
    )
    return fn(ftab, idx4)


def kernel(x, tables):
    ftab = tables.reshape(F * V, D)
    idx4 = x.reshape(NW, NJ, CH, F).transpose(0, 3, 1, 2)
    out_flat = _encode(ftab, idx4)
    return out_flat.reshape(B, D)


# speedup vs baseline: 1.1308x; 1.1308x over previous
import jax
import jax.numpy as jnp
from jax import lax
from jax.experimental import pallas as pl
from jax.experimental.pallas import tpu as pltpu
from jax.experimental.pallas import tpu_sc as plsc

B = 16384
F = 26
V = 100000
D = 32

NC = 2
NS = 16
NW = NC * NS
BPW = B // NW
CH = 128
NJ = BPW // CH
L = 16


def _body(ftab_hbm, idx_hbm, out_hbm, idx_v, buf_v, acc_v, sem_a, sem0, sem1):
    c = lax.axis_index("c")
    s = lax.axis_index("s")
    wid = s * NC + c

    pltpu.sync_copy(idx_hbm.at[wid], idx_v)

    def offs_f(f, _):
        @plsc.parallel_loop(0, BPW // L, unroll=8)
        def _ol(i):
            j = i // (CH // L)
            cc = i - j * (CH // L)
            sl = idx_v.at[f, j, pl.ds(cc * L, L)]
            sl[...] = sl[...] + f * V
        return 0
    lax.fori_loop(1, F, offs_f, 0)

    sems = (sem0, sem1)

    def fire(f, slot):
        for j in range(NJ):
            pltpu.async_copy(ftab_hbm.at[idx_v.at[f, j]], buf_v.at[slot, j],
                             sems[slot])

    def drain(slot):
        for j in range(NJ):
            pltpu.make_async_copy(ftab_hbm.at[idx_v.at[0, j]],
                                  buf_v.at[slot, j], sems[slot]).wait()

    def accumulate(slot):
        @plsc.parallel_loop(0, BPW, unroll=4)
        def _al(r):
            j = r // CH
            rr = r - j * CH
            for h in range(D // L):
                plsc.addupdate(acc_v.at[j, rr, pl.ds(h * L, L)],
                               buf_v[slot, j, rr, pl.ds(h * L, L)])

    for j in range(NJ):
        pltpu.async_copy(ftab_hbm.at[idx_v.at[0, j]], acc_v.at[j], sem_a)
    fire(1, 0)
    for j in range(NJ):
        pltpu.make_async_copy(ftab_hbm.at[idx_v.at[0, j]], acc_v.at[j],
                              sem_a).wait()

    def step(i, _):
        fa = 2 * i + 1
        fire(fa + 1, 1)
        drain(0)
        accumulate(0)

        fire(fa + 2, 0)
        drain(1)
        accumulate(1)
        return 0
    lax.fori_loop(0, (F - 2) // 2, step, 0)

    drain(0)
    accumulate(0)

    pltpu.sync_copy(acc_v, out_hbm.at[wid])


@jax.jit
def _encode(ftab, idx4):
    mesh = plsc.VectorSubcoreMesh(
        core_axis_name="c", subcore_axis_name="s",
        num_cores=NC, num_subcores=NS)
    fn = pl.kernel(
        _body,
        out_type=jax.ShapeDtypeStruct((NW, NJ, CH, D), jnp.float32),
        mesh=mesh,
        scratch_types=[
            pltpu.VMEM((F, NJ, CH), jnp.int32),
            pltpu.VMEM((2, NJ, CH, D), jnp.float32),
            pltpu.VMEM((NJ, CH, D), jnp.float32),
            pltpu.SemaphoreType.DMA,
            pltpu.SemaphoreType.DMA,
            pltpu.SemaphoreType.DMA,
        ],
        compiler_params=pltpu.CompilerParams(use_tc_tiling_on_sc=False),
    )
    return fn(ftab, idx4)


def kernel(x, tables):
    ftab = tables.reshape(F * V, D)
    idx4 = x.reshape(NW, NJ, CH, F).transpose(0, 3, 1, 2)
    out4 = _encode(ftab, idx4)
    return out4.reshape(B, D)


# speedup vs baseline: 1.6623x; 1.4700x over previous
import jax
import jax.numpy as jnp
from jax import lax
from jax.experimental import pallas as pl
from jax.experimental.pallas import tpu as pltpu
from jax.experimental.pallas import tpu_sc as plsc

B = 16384
F = 26
V = 100000
D = 32

NC = 2
NS = 16
NW = NC * NS
BPW = B // NW
CH = 128
NJ = BPW // CH
L = 16
G = V // 4
NSTEP = F * NJ
HR = 5000


def _body(tab_hbm, idx_hbm, out_hbm, idx_v, buf_v, acc_v, sem0, sem1):
    c = lax.axis_index("c")
    s = lax.axis_index("s")
    wid = s * NC + c

    pltpu.sync_copy(idx_hbm.at[wid], idx_v)

    def prep(i, _):
        f = i // (BPW // L)
        r = i - f * (BPW // L)
        j = r // (CH // L)
        cc = r - j * (CH // L)
        sl = (f, j, pl.ds(cc * L, L))
        v = idx_v[sl]
        q = v // G
        idx_v[sl] = (f * G + v - q * G) * 4 + q
        return 0
    lax.fori_loop(0, F * BPW // L, prep, 0)

    def zstep(i, _):
        acc_v[pl.ds(i * L, L)] = jnp.zeros((L,), jnp.float32)
        return 0
    lax.fori_loop(0, BPW * D // L, zstep, 0)

    sems = (sem0, sem1)

    def fire(g, slot):
        f = g // NJ
        j = g - f * NJ
        pltpu.async_copy(tab_hbm.at[idx_v.at[f, j]], buf_v.at[slot],
                         sems[slot])

    def drain(slot):
        pltpu.make_async_copy(tab_hbm.at[idx_v.at[0, 0]], buf_v.at[slot],
                              sems[slot]).wait()

    def accumulate(g, slot):
        f = g // NJ
        j = g - f * NJ
        jbase = j * CH * D

        @plsc.parallel_loop(0, CH, unroll=4)
        def _al(r):
            for h in range(D // L):
                plsc.addupdate(acc_v.at[pl.ds(jbase + r * D + h * L, L)],
                               buf_v[slot, r, pl.ds(h * L, L)])

    fire(0, 0)

    def step(i, _):
        g0 = 2 * i
        fire(g0 + 1, 1)
        drain(0)
        accumulate(g0, 0)

        @pl.when(g0 + 2 < NSTEP)
        def _():
            fire(g0 + 2, 0)
        drain(1)
        accumulate(g0 + 1, 1)
        return 0
    lax.fori_loop(0, NSTEP // 2, step, 0)

    pltpu.sync_copy(acc_v, out_hbm.at[pl.ds(wid * BPW * D, BPW * D)])


def _tc_flatten_body(in_ref, out_ref):
    for h in range(G // HR):
        parts = [
            in_ref[0, :, s * G + h * HR:s * G + (h + 1) * HR].T
            for s in range(4)
        ]
        out_ref[h * HR:(h + 1) * HR, :] = jnp.concatenate(parts, axis=1)


def _tc_flatten(tabt):
    return pl.pallas_call(
        _tc_flatten_body,
        grid=(F,),
        in_specs=[pl.BlockSpec((1, D, V), lambda f: (f, 0, 0))],
        out_specs=pl.BlockSpec((V // 4, 128), lambda f: (f, 0)),
        out_shape=jax.ShapeDtypeStruct((F * V // 4, 128), jnp.float32),
        compiler_params=pltpu.CompilerParams(
            vmem_limit_bytes=60000 * 1024),
    )(tabt)


@jax.jit
def _encode(tab, idx4):
    mesh = plsc.VectorSubcoreMesh(
        core_axis_name="c", subcore_axis_name="s",
        num_cores=NC, num_subcores=NS)
    fn = pl.kernel(
        _body,
        out_type=jax.ShapeDtypeStruct((B * D,), jnp.float32),
        mesh=mesh,
        scratch_types=[
            pltpu.VMEM((F, NJ, CH), jnp.int32),
            pltpu.VMEM((2, CH, D), jnp.float32),
            pltpu.VMEM((BPW * D,), jnp.float32),
            pltpu.SemaphoreType.DMA,
            pltpu.SemaphoreType.DMA,
        ],
        compiler_params=pltpu.CompilerParams(use_tc_tiling_on_sc=False,
                                             needs_layout_passes=False),
    )
    return fn(tab, idx4)


def kernel(x, tables):
    tab128 = _tc_flatten(tables.transpose(0, 2, 1))
    tab = tab128.reshape(F * V, D)
    idx4 = x.reshape(NW, NJ, CH, F).transpose(0, 3, 1, 2)
    out_flat = _encode(tab, idx4)
    return out_flat.reshape(B, D)


# speedup vs baseline: 1.8103x; 1.0890x over previous
import jax
import jax.numpy as jnp
from jax import lax
from jax.experimental import pallas as pl
from jax.experimental.pallas import tpu as pltpu
from jax.experimental.pallas import tpu_sc as plsc

B = 16384
F = 26
V = 100000
D = 32

NC = 2
NS = 16
NW = NC * NS
BPW = B // NW
CH = 128
NJ = BPW // CH
L = 16
NSTEP = F * NJ
Q = 24960
GP = V - 3 * Q
TCW = 2560


def _body(tab_hbm, idx_hbm, out_hbm, idx_v, buf_v, acc_v, sem0, sem1):
    c = lax.axis_index("c")
    s = lax.axis_index("s")
    wid = s * NC + c

    pltpu.sync_copy(idx_hbm.at[wid], idx_v)

    def prep(i, _):
        f = i // (BPW // L)
        r = i - f * (BPW // L)
        j = r // (CH // L)
        cc = r - j * (CH // L)
        sl = (f, j, pl.ds(cc * L, L))
        v = idx_v[sl]
        q = jnp.minimum(v // Q, 3)
        idx_v[sl] = (f * GP + v - q * Q) * 4 + q
        return 0
    lax.fori_loop(0, F * BPW // L, prep, 0)

    def zstep(i, _):
        acc_v[pl.ds(i * L, L)] = jnp.zeros((L,), jnp.float32)
        return 0
    lax.fori_loop(0, BPW * D // L, zstep, 0)

    sems = (sem0, sem1)

    def fire(g, slot):
        f = g // NJ
        j = g - f * NJ
        pltpu.async_copy(tab_hbm.at[idx_v.at[f, j]], buf_v.at[slot],
                         sems[slot])

    def drain(slot):
        pltpu.make_async_copy(tab_hbm.at[idx_v.at[0, 0]], buf_v.at[slot],
                              sems[slot]).wait()

    def accumulate(g, slot):
        f = g // NJ
        j = g - f * NJ
        jbase = j * CH * D

        @plsc.parallel_loop(0, CH, unroll=4)
        def _al(r):
            for h in range(D // L):
                plsc.addupdate(acc_v.at[pl.ds(jbase + r * D + h * L, L)],
                               buf_v[slot, r, pl.ds(h * L, L)])

    fire(0, 0)

    def step(i, _):
        g0 = 2 * i
        fire(g0 + 1, 1)
        drain(0)
        accumulate(g0, 0)

        @pl.when(g0 + 2 < NSTEP)
        def _():
            fire(g0 + 2, 0)
        drain(1)
        accumulate(g0 + 1, 1)
        return 0
    lax.fori_loop(0, NSTEP // 2, step, 0)

    pltpu.sync_copy(acc_v, out_hbm.at[pl.ds(wid * BPW * D, BPW * D)])


def _tc_flatten_body(in_ref, out_ref):
    def win(h, _):
        r0 = h * TCW
        parts = [
            in_ref[0, :, pl.ds(s * Q + r0, TCW)].T
            for s in range(4)
        ]
        out_ref[pl.ds(r0, TCW), :] = jnp.concatenate(parts, axis=1)
        return 0
    nfull = (GP // TCW)
    lax.fori_loop(0, nfull, win, 0)
    r0 = nfull * TCW
    wt = GP - r0
    parts = [in_ref[0, :, s * Q + r0:s * Q + r0 + wt].T for s in range(4)]
    out_ref[r0:r0 + wt, :] = jnp.concatenate(parts, axis=1)


def _tc_flatten(tabt):
    return pl.pallas_call(
        _tc_flatten_body,
        grid=(F,),
        in_specs=[pl.BlockSpec((1, D, V), lambda f: (f, 0, 0))],
        out_specs=pl.BlockSpec((GP, 128), lambda f: (f, 0)),
        out_shape=jax.ShapeDtypeStruct((F * GP, 128), jnp.float32),
        compiler_params=pltpu.CompilerParams(
            vmem_limit_bytes=60000 * 1024),
    )(tabt)


@jax.jit
def _encode(tab, idx4):
    mesh = plsc.VectorSubcoreMesh(
        core_axis_name="c", subcore_axis_name="s",
        num_cores=NC, num_subcores=NS)
    fn = pl.kernel(
        _body,
        out_type=jax.ShapeDtypeStruct((B * D,), jnp.float32),
        mesh=mesh,
        scratch_types=[
            pltpu.VMEM((F, NJ, CH), jnp.int32),
            pltpu.VMEM((2, CH, D), jnp.float32),
            pltpu.VMEM((BPW * D,), jnp.float32),
            pltpu.SemaphoreType.DMA,
            pltpu.SemaphoreType.DMA,
        ],
        compiler_params=pltpu.CompilerParams(use_tc_tiling_on_sc=False,
                                             needs_layout_passes=False),
    )
    return fn(tab, idx4)


def kernel(x, tables):
    tab128 = _tc_flatten(tables.transpose(0, 2, 1))
    tab = tab128.reshape(F * GP * 4, D)
    idx4 = x.reshape(NW, NJ, CH, F).transpose(0, 3, 1, 2)
    out_flat = _encode(tab, idx4)
    return out_flat.reshape(B, D)
